# baseline (speedup 1.0000x reference)
import jax
import jax.numpy as jnp
from jax import lax
from jax.experimental import pallas as pl
from jax.experimental.pallas import tpu as pltpu
from jax.experimental.pallas import tpu_sc as plsc

N_NODES = 10000
N_EDGES = 160000
D_FEAT = 256
N_SUPER = 512

NC = 2
NS = 16
L = 16

EPC = N_EDGES // NC
EPT = EPC // NS
NITER = (EPT + L - 1) // L
K = 128
COMP = ((EPT + K - 1) // K) * K
PCH = 512
SPT = N_SUPER // NS
OWN_CAP = 768
ACC_SEG = SPT + 8
ROWW = D_FEAT + 128


def _main_body(emb_hbm, src_hbm, dst_hbm, remap_hbm, zacc_hbm, canon_hbm,
               part_hbm, out_hbm,
               remap_v, dst_v, src_v, comp_src, comp_seg, seg_arr, pc_arr,
               scan_src, scan_seg, own_src, own_seg,
               srcbuf, rows_v, cnt_all_v, cntpub_v,
               acc2d, canon_v, canonb_v, r0, r1, outb,
               pair_src_sh, pair_seg_sh, cntp_sh, sem, sem2, xsem):
    cid = lax.axis_index("c")
    sid = lax.axis_index("s")
    lane = lax.iota(jnp.int32, L)

    ebase = cid * EPC + sid * EPT
    c0 = pltpu.async_copy(remap_hbm, remap_v, sem)
    c1 = pltpu.async_copy(src_hbm.at[pl.ds(ebase, EPT)],
                          src_v.at[pl.ds(0, EPT)], sem)
    c2 = pltpu.async_copy(dst_hbm.at[pl.ds(ebase, EPT)],
                          dst_v.at[pl.ds(0, EPT)], sem)
    c3 = pltpu.async_copy(zacc_hbm, acc2d, sem)
    c0.wait()
    c1.wait()
    c2.wait()
    c3.wait()

    tail = EPT - (EPT % L) if EPT % L else EPT
    if EPT % L:
        v = dst_v[pl.ds(tail, L)]
        dst_v[pl.ds(tail, L)] = jnp.where(lane < (EPT - tail), v, N_NODES)

    lane0 = lane == 0

    @plsc.parallel_loop(0, NITER, unroll=4)
    def _passA(i):
        dstv = dst_v[pl.ds(i * L, L)]
        segv = plsc.load_gather(remap_v, [dstv])
        seg_arr[pl.ds(i * L, L)] = segv
        pc = plsc.all_reduce_population_count(segv >= 0)
        plsc.store_scatter(pc_arr, [jnp.full((L,), 0, jnp.int32) + i], pc,
                           mask=lane0)

    ptail = (NITER // L) * L
    if NITER % L:
        pv = pc_arr[pl.ds(ptail, L)]
        pc_arr[pl.ds(ptail, L)] = jnp.where(lane < (NITER - ptail), pv, 0)

    def pfx(g, running):
        pcv = pc_arr[pl.ds(g * L, L)]
        cums = plsc.cumsum(pcv)
        pc_arr[pl.ds(g * L, L)] = running + (cums - pcv)
        return running + cums[L - 1]

    cnt = lax.fori_loop(0, (NITER + L - 1) // L, pfx, jnp.int32(0))

    @plsc.parallel_loop(0, NITER, unroll=4)
    def _passC(i):
        offv = plsc.load_gather(pc_arr, [jnp.full((L,), 0, jnp.int32) + i])
        off_s = offv[0]
        segv = seg_arr[pl.ds(i * L, L)]
        srcv = src_v[pl.ds(i * L, L)]
        m = segv >= 0
        plsc.store_compressed(comp_seg.at[pl.ds(off_s, L)], segv, mask=m)
        plsc.store_compressed(comp_src.at[pl.ds(off_s, L)], srcv, mask=m)

    pidx = cnt + lane
    plsc.store_scatter(comp_seg, [pidx], jnp.full((L,), N_SUPER, jnp.int32),
                       mask=pidx < ((cnt + L - 1) // L) * L)

    cntpub_v[pl.ds(0, L)] = jnp.full((L,), 0, jnp.int32) + cnt
    pltpu.sync_copy(cntpub_v, cntp_sh.at[pl.ds(sid * L, L)])
    npub = (cnt + PCH - 1) // PCH

    def pbody(j, _):
        off = pl.multiple_of(j * PCH, PCH)
        pltpu.sync_copy(comp_src.at[pl.ds(off, PCH)],
                        pair_src_sh.at[pl.ds(sid * COMP + off, PCH)])
        pltpu.sync_copy(comp_seg.at[pl.ds(off, PCH)],
                        pair_seg_sh.at[pl.ds(sid * COMP + off, PCH)])
        return 0

    lax.fori_loop(0, npub, pbody, 0)
    plsc.subcore_barrier()

    own_lo = sid * SPT
    pltpu.sync_copy(cntp_sh, cnt_all_v)
    ones16 = jnp.ones((L,), jnp.float32)

    def flush16(bbase):
        srcbuf[pl.ds(0, L)] = own_src[pl.ds(bbase, L)]
        pltpu.async_copy(emb_hbm.at[srcbuf.at[pl.ds(0, L)]],
                         rows_v.at[pl.ds(0, L)], sem).wait()

        def rb16(r, _):
            seg_splat = plsc.load_gather(
                own_seg, [jnp.full((L,), 0, jnp.int32) + (bbase + r)])

            @plsc.parallel_loop(0, D_FEAT // L, unroll=D_FEAT // L)
            def _chunks(ch):
                plsc.addupdate_scatter(acc2d, [seg_splat, lane + ch * L],
                                       rows_v[r, pl.ds(ch * L, L)])

            plsc.addupdate_scatter(acc2d, [seg_splat, lane + D_FEAT], ones16)
            return 0

        lax.fori_loop(0, L, rb16, 0)

    def flush_blocks(ocnt):
        nb = ocnt // L

        def fb(b, _):
            flush16(pl.multiple_of(b * L, L))
            return 0

        lax.fori_loop(0, nb, fb, 0)
        rem = ocnt - nb * L
        rbase = pl.multiple_of(nb * L, L)
        own_src[pl.ds(0, L)] = own_src[pl.ds(rbase, L)]
        own_seg[pl.ds(0, L)] = own_seg[pl.ds(rbase, L)]
        return rem

    def fire_prefetch(t, p):
        poff = pl.multiple_of(t * COMP, PCH)
        boff = pl.multiple_of(p * PCH, PCH)
        pltpu.async_copy(pair_src_sh.at[pl.ds(poff, PCH)],
                         scan_src.at[pl.ds(boff, PCH)], sem2)
        pltpu.async_copy(pair_seg_sh.at[pl.ds(poff, PCH)],
                         scan_seg.at[pl.ds(boff, PCH)], sem2)

    def drain_prefetch():
        pltpu.make_async_copy(pair_src_sh.at[pl.ds(0, PCH)],
                              scan_src.at[pl.ds(0, PCH)], sem2).wait()
        pltpu.make_async_copy(pair_seg_sh.at[pl.ds(0, PCH)],
                              scan_seg.at[pl.ds(0, PCH)], sem2).wait()

    fire_prefetch(jnp.int32(0), jnp.int32(0))

    def tbody(t, ocnt):
        nt = cnt_all_v[pl.ds(t * L, L)][0]
        nchk = (nt + PCH - 1) // PCH
        p = lax.rem(t, 2)
        drain_prefetch()
        fire_prefetch(jnp.minimum(t + 1, NS - 1), lax.rem(t + 1, 2))
        base = pl.multiple_of(p * PCH, PCH)

        def jbody(j, ocnt):
            @pl.when(j > 0)
            def _extra_chunk():
                poff = pl.multiple_of(t * COMP + j * PCH, PCH)
                pltpu.sync_copy(pair_src_sh.at[pl.ds(poff, PCH)],
                                scan_src.at[pl.ds(base, PCH)])
                pltpu.sync_copy(pair_seg_sh.at[pl.ds(poff, PCH)],
                                scan_seg.at[pl.ds(base, PCH)])

            def kbody(k, ocnt):
                segv = scan_seg[pl.ds(base + k * L, L)]
                srcv = scan_src[pl.ds(base + k * L, L)]
                m = (segv >= own_lo) & (segv < own_lo + SPT)
                plsc.store_compressed(own_seg.at[pl.ds(ocnt, L)],
                                      segv - own_lo, mask=m)
                plsc.store_compressed(own_src.at[pl.ds(ocnt, L)],
                                      srcv, mask=m)
                return ocnt + plsc.all_reduce_population_count(m)[0]

            nvr = (jnp.minimum(nt - j * PCH, PCH) + L - 1) // L
            ocnt = lax.fori_loop(0, nvr, kbody, ocnt)
            return flush_blocks(ocnt)

        return lax.fori_loop(0, nchk, jbody, ocnt)

    ocnt = lax.fori_loop(0, NS, tbody, jnp.int32(0))
    drain_prefetch()

    pend = ((ocnt + L - 1) // L) * L
    idx = ocnt + lane
    plsc.store_scatter(own_seg, [idx], jnp.full((L,), SPT, jnp.int32),
                       mask=idx < pend)
    plsc.store_scatter(own_src, [idx], jnp.zeros((L,), jnp.int32),
                       mask=idx < pend)
    flush_blocks(pend)

    pltpu.sync_copy(acc2d.at[pl.ds(0, SPT)],
                    part_hbm.at[pl.ds(cid * N_SUPER + sid * SPT, SPT)])

    plsc.subcore_barrier()
    pl.semaphore_signal(xsem, 1, core_index=1 - cid)
    pl.semaphore_wait(xsem, 1)

    wid = sid * NC + cid
    base = wid * L
    pltpu.sync_copy(canon_hbm.at[pl.ds(base, L)], canon_v)
    cv = canon_v[pl.ds(0, L)]
    canonb_v[pl.ds(0, L)] = cv + N_SUPER

    g0 = pltpu.async_copy(part_hbm.at[canon_v], r0, sem)
    g1 = pltpu.async_copy(part_hbm.at[canonb_v], r1, sem)
    g0.wait()
    g1.wait()

    one = jnp.ones((L,), jnp.float32)

    def rbody(r, _):
        cntv = r0[r, pl.ds(D_FEAT, L)] + r1[r, pl.ds(D_FEAT, L)]
        scale = one / jnp.maximum(cntv, one)

        def chb(ch, _):
            o = pl.multiple_of(ch * L, L)
            outb[r, pl.ds(o, L)] = (r0[r, pl.ds(o, L)]
                                    + r1[r, pl.ds(o, L)]) * scale
            return 0

        lax.fori_loop(0, D_FEAT // L, chb, 0)
        return 0

    lax.fori_loop(0, L, rbody, 0)
    pltpu.sync_copy(outb, out_hbm.at[pl.ds(base, L)])


def _make_mesh():
    return plsc.VectorSubcoreMesh(core_axis_name="c", subcore_axis_name="s",
                                  num_cores=NC, num_subcores=NS)


_main = pl.kernel(
    _main_body,
    out_type=(
        jax.ShapeDtypeStruct((NC * N_SUPER, ROWW), jnp.float32),
        jax.ShapeDtypeStruct((N_SUPER, D_FEAT), jnp.float32),
    ),
    mesh=_make_mesh(),
    compiler_params=pltpu.CompilerParams(needs_layout_passes=False),
    scratch_types=[
        pltpu.VMEM((N_NODES + L,), jnp.int32),
        pltpu.VMEM((EPT + 8,), jnp.int32),
        pltpu.VMEM((EPT + 8,), jnp.int32),
        pltpu.VMEM((COMP,), jnp.int32),
        pltpu.VMEM((COMP,), jnp.int32),
        pltpu.VMEM((COMP,), jnp.int32),
        pltpu.VMEM((((NITER + L - 1) // L) * L,), jnp.int32),
        pltpu.VMEM((2 * PCH,), jnp.int32),
        pltpu.VMEM((2 * PCH,), jnp.int32),
        pltpu.VMEM((OWN_CAP,), jnp.int32),
        pltpu.VMEM((OWN_CAP,), jnp.int32),
        pltpu.VMEM((K,), jnp.int32),
        pltpu.VMEM((K, D_FEAT), jnp.float32),
        pltpu.VMEM((NS * L,), jnp.int32),
        pltpu.VMEM((L,), jnp.int32),
        pltpu.VMEM((ACC_SEG, ROWW), jnp.float32),
        pltpu.VMEM((L,), jnp.int32),
        pltpu.VMEM((L,), jnp.int32),
        pltpu.VMEM((L, ROWW), jnp.float32),
        pltpu.VMEM((L, ROWW), jnp.float32),
        pltpu.VMEM((L, D_FEAT), jnp.float32),
        pltpu.VMEM_SHARED((NS * COMP,), jnp.int32),
        pltpu.VMEM_SHARED((NS * COMP,), jnp.int32),
        pltpu.VMEM_SHARED((NS * L,), jnp.int32),
        pltpu.SemaphoreType.DMA,
        pltpu.SemaphoreType.DMA,
        pltpu.SemaphoreType.REGULAR,
    ],
)

@jax.jit
def kernel(all_node_emb, supernode_edge_index, supernode_idx, graph_batch):
    emb = all_node_emb.astype(jnp.float32)
    src = supernode_edge_index[0].astype(jnp.int32)
    dst = supernode_edge_index[1].astype(jnp.int32)
    snode = supernode_idx.astype(jnp.int32)

    remap = jnp.full((N_NODES + L,), -1, jnp.int32).at[snode].set(
        jnp.arange(N_SUPER, dtype=jnp.int32))
    canon = remap[snode]

    zacc = jnp.zeros((ACC_SEG, ROWW), jnp.float32)

    _, out = _main(emb, src, dst, remap, zacc, canon)
    return out


# speedup vs baseline: 1.0690x; 1.0690x over previous
import jax
import jax.numpy as jnp
from jax import lax
from jax.experimental import pallas as pl
from jax.experimental.pallas import tpu as pltpu
from jax.experimental.pallas import tpu_sc as plsc

N_NODES = 10000
N_EDGES = 160000
D_FEAT = 256
N_SUPER = 512

NC = 2
NS = 16
L = 16

EPC = N_EDGES // NC
EPT = EPC // NS
NITER = (EPT + L - 1) // L
K = 128
COMP = ((EPT + K - 1) // K) * K
PCH = 512
SPT = N_SUPER // NS
OWN_CAP = 768
ACC_SEG = SPT + 8
ROWW = D_FEAT + 128


def _main_body(emb_hbm, src_hbm, dst_hbm, remap_hbm, zacc_hbm, canon_hbm,
               part_hbm, out_hbm,
               remap_v, dst_v, src_v, comp_src, comp_seg, seg_arr, pc_arr,
               scan_src, scan_seg, own_src, own_seg,
               srcbuf, rows_v, cnt_all_v, cntpub_v,
               acc2d, canon_v, canonb_v, r0, r1, outb,
               pair_src_sh, pair_seg_sh, cntp_sh, sem, sem2, xsem):
    cid = lax.axis_index("c")
    sid = lax.axis_index("s")
    lane = lax.iota(jnp.int32, L)

    ebase = cid * EPC + sid * EPT
    c0 = pltpu.async_copy(remap_hbm, remap_v, sem)
    c1 = pltpu.async_copy(src_hbm.at[pl.ds(ebase, EPT)],
                          src_v.at[pl.ds(0, EPT)], sem)
    c2 = pltpu.async_copy(dst_hbm.at[pl.ds(ebase, EPT)],
                          dst_v.at[pl.ds(0, EPT)], sem)
    c3 = pltpu.async_copy(zacc_hbm, acc2d, sem)
    c0.wait()
    c1.wait()
    c2.wait()
    c3.wait()

    tail = EPT - (EPT % L) if EPT % L else EPT
    if EPT % L:
        v = dst_v[pl.ds(tail, L)]
        dst_v[pl.ds(tail, L)] = jnp.where(lane < (EPT - tail), v, N_NODES)

    lane0 = lane == 0

    @plsc.parallel_loop(0, NITER, unroll=4)
    def _passA(i):
        dstv = dst_v[pl.ds(i * L, L)]
        segv = plsc.load_gather(remap_v, [dstv])
        seg_arr[pl.ds(i * L, L)] = segv
        pc = plsc.all_reduce_population_count(segv >= 0)
        plsc.store_scatter(pc_arr, [jnp.full((L,), 0, jnp.int32) + i], pc,
                           mask=lane0)

    ptail = (NITER // L) * L
    if NITER % L:
        pv = pc_arr[pl.ds(ptail, L)]
        pc_arr[pl.ds(ptail, L)] = jnp.where(lane < (NITER - ptail), pv, 0)

    def pfx(g, running):
        pcv = pc_arr[pl.ds(g * L, L)]
        cums = plsc.cumsum(pcv)
        pc_arr[pl.ds(g * L, L)] = running + (cums - pcv)
        return running + cums[L - 1]

    cnt = lax.fori_loop(0, (NITER + L - 1) // L, pfx, jnp.int32(0))

    @plsc.parallel_loop(0, NITER, unroll=4)
    def _passC(i):
        offv = plsc.load_gather(pc_arr, [jnp.full((L,), 0, jnp.int32) + i])
        off_s = offv[0]
        segv = seg_arr[pl.ds(i * L, L)]
        srcv = src_v[pl.ds(i * L, L)]
        m = segv >= 0
        plsc.store_compressed(comp_seg.at[pl.ds(off_s, L)], segv, mask=m)
        plsc.store_compressed(comp_src.at[pl.ds(off_s, L)], srcv, mask=m)

    pidx = cnt + lane
    plsc.store_scatter(comp_seg, [pidx], jnp.full((L,), N_SUPER, jnp.int32),
                       mask=pidx < ((cnt + L - 1) // L) * L)

    cntpub_v[pl.ds(0, L)] = jnp.full((L,), 0, jnp.int32) + cnt
    pltpu.sync_copy(cntpub_v, cntp_sh.at[pl.ds(sid * L, L)])
    npub = (cnt + PCH - 1) // PCH

    def pbody(j, _):
        off = pl.multiple_of(j * PCH, PCH)
        pltpu.sync_copy(comp_src.at[pl.ds(off, PCH)],
                        pair_src_sh.at[pl.ds(sid * COMP + off, PCH)])
        pltpu.sync_copy(comp_seg.at[pl.ds(off, PCH)],
                        pair_seg_sh.at[pl.ds(sid * COMP + off, PCH)])
        return 0

    lax.fori_loop(0, npub, pbody, 0)
    plsc.subcore_barrier()

    own_lo = sid * SPT
    pltpu.sync_copy(cntp_sh, cnt_all_v)
    ones16 = jnp.ones((L,), jnp.float32)

    def flush_blocks(ocnt):
        nfl = ocnt // K

        def fb(b, _):
            bbase = pl.multiple_of(b * K, K)
            for j in range(K // L):
                srcbuf[pl.ds(j * L, L)] = own_src[pl.ds(bbase + j * L, L)]
            pltpu.async_copy(emb_hbm.at[srcbuf], rows_v, sem).wait()

            def rb(r, _):
                seg_splat = plsc.load_gather(
                    own_seg, [jnp.full((L,), 0, jnp.int32) + (bbase + r)])

                @plsc.parallel_loop(0, D_FEAT // L, unroll=D_FEAT // L)
                def _chunks(ch):
                    plsc.addupdate_scatter(acc2d, [seg_splat, lane + ch * L],
                                           rows_v[r, pl.ds(ch * L, L)])

                plsc.addupdate_scatter(acc2d, [seg_splat, lane + D_FEAT],
                                       ones16)
                return 0

            lax.fori_loop(0, K, rb, 0)
            return 0

        lax.fori_loop(0, nfl, fb, 0)
        rem = ocnt - nfl * K
        rbase = pl.multiple_of(nfl * K, K)
        for j in range(K // L):
            own_src[pl.ds(j * L, L)] = own_src[pl.ds(rbase + j * L, L)]
            own_seg[pl.ds(j * L, L)] = own_seg[pl.ds(rbase + j * L, L)]
        return rem

    def fire_prefetch(t, p):
        poff = pl.multiple_of(t * COMP, PCH)
        boff = pl.multiple_of(p * PCH, PCH)
        pltpu.async_copy(pair_src_sh.at[pl.ds(poff, PCH)],
                         scan_src.at[pl.ds(boff, PCH)], sem2)
        pltpu.async_copy(pair_seg_sh.at[pl.ds(poff, PCH)],
                         scan_seg.at[pl.ds(boff, PCH)], sem2)

    def drain_prefetch():
        pltpu.make_async_copy(pair_src_sh.at[pl.ds(0, PCH)],
                              scan_src.at[pl.ds(0, PCH)], sem2).wait()
        pltpu.make_async_copy(pair_seg_sh.at[pl.ds(0, PCH)],
                              scan_seg.at[pl.ds(0, PCH)], sem2).wait()

    fire_prefetch(jnp.int32(0), jnp.int32(0))

    def tbody(t, ocnt):
        nt = cnt_all_v[pl.ds(t * L, L)][0]
        nchk = (nt + PCH - 1) // PCH
        p = lax.rem(t, 2)
        drain_prefetch()
        fire_prefetch(jnp.minimum(t + 1, NS - 1), lax.rem(t + 1, 2))
        base = pl.multiple_of(p * PCH, PCH)

        def jbody(j, ocnt):
            @pl.when(j > 0)
            def _extra_chunk():
                poff = pl.multiple_of(t * COMP + j * PCH, PCH)
                pltpu.sync_copy(pair_src_sh.at[pl.ds(poff, PCH)],
                                scan_src.at[pl.ds(base, PCH)])
                pltpu.sync_copy(pair_seg_sh.at[pl.ds(poff, PCH)],
                                scan_seg.at[pl.ds(base, PCH)])

            def kbody(k, ocnt):
                segv = scan_seg[pl.ds(base + k * L, L)]
                srcv = scan_src[pl.ds(base + k * L, L)]
                m = (segv >= own_lo) & (segv < own_lo + SPT)
                plsc.store_compressed(own_seg.at[pl.ds(ocnt, L)],
                                      segv - own_lo, mask=m)
                plsc.store_compressed(own_src.at[pl.ds(ocnt, L)],
                                      srcv, mask=m)
                return ocnt + plsc.all_reduce_population_count(m)[0]

            nvr = (jnp.minimum(nt - j * PCH, PCH) + L - 1) // L
            ocnt = lax.fori_loop(0, nvr, kbody, ocnt)
            return flush_blocks(ocnt)

        return lax.fori_loop(0, nchk, jbody, ocnt)

    ocnt = lax.fori_loop(0, NS, tbody, jnp.int32(0))
    drain_prefetch()

    pend = ((ocnt + L - 1) // L) * L
    idx = ocnt + lane
    plsc.store_scatter(own_seg, [idx], jnp.full((L,), SPT, jnp.int32),
                       mask=idx < pend)
    plsc.store_scatter(own_src, [idx], jnp.zeros((L,), jnp.int32),
                       mask=idx < pend)

    def tb(b, _):
        bbase = pl.multiple_of(b * L, L)
        srcbuf[pl.ds(0, L)] = own_src[pl.ds(bbase, L)]
        pltpu.async_copy(emb_hbm.at[srcbuf.at[pl.ds(0, L)]],
                         rows_v.at[pl.ds(0, L)], sem).wait()

        def rb16(r, _):
            seg_splat = plsc.load_gather(
                own_seg, [jnp.full((L,), 0, jnp.int32) + (bbase + r)])

            @plsc.parallel_loop(0, D_FEAT // L, unroll=D_FEAT // L)
            def _chunks(ch):
                plsc.addupdate_scatter(acc2d, [seg_splat, lane + ch * L],
                                       rows_v[r, pl.ds(ch * L, L)])

            plsc.addupdate_scatter(acc2d, [seg_splat, lane + D_FEAT], ones16)
            return 0

        lax.fori_loop(0, L, rb16, 0)
        return 0

    lax.fori_loop(0, pend // L, tb, 0)

    pltpu.sync_copy(acc2d.at[pl.ds(0, SPT)],
                    part_hbm.at[pl.ds(cid * N_SUPER + sid * SPT, SPT)])

    plsc.subcore_barrier()
    pl.semaphore_signal(xsem, 1, core_index=1 - cid)
    pl.semaphore_wait(xsem, 1)

    wid = sid * NC + cid
    base = wid * L
    pltpu.sync_copy(canon_hbm.at[pl.ds(base, L)], canon_v)
    cv = canon_v[pl.ds(0, L)]
    canonb_v[pl.ds(0, L)] = cv + N_SUPER

    g0 = pltpu.async_copy(part_hbm.at[canon_v], r0, sem)
    g1 = pltpu.async_copy(part_hbm.at[canonb_v], r1, sem)
    g0.wait()
    g1.wait()

    one = jnp.ones((L,), jnp.float32)

    def rbody(r, _):
        cntv = r0[r, pl.ds(D_FEAT, L)] + r1[r, pl.ds(D_FEAT, L)]
        scale = one / jnp.maximum(cntv, one)

        def chb(ch, _):
            o = pl.multiple_of(ch * L, L)
            outb[r, pl.ds(o, L)] = (r0[r, pl.ds(o, L)]
                                    + r1[r, pl.ds(o, L)]) * scale
            return 0

        lax.fori_loop(0, D_FEAT // L, chb, 0)
        return 0

    lax.fori_loop(0, L, rbody, 0)
    pltpu.sync_copy(outb, out_hbm.at[pl.ds(base, L)])


def _make_mesh():
    return plsc.VectorSubcoreMesh(core_axis_name="c", subcore_axis_name="s",
                                  num_cores=NC, num_subcores=NS)


_main = pl.kernel(
    _main_body,
    out_type=(
        jax.ShapeDtypeStruct((NC * N_SUPER, ROWW), jnp.float32),
        jax.ShapeDtypeStruct((N_SUPER, D_FEAT), jnp.float32),
    ),
    mesh=_make_mesh(),
    compiler_params=pltpu.CompilerParams(needs_layout_passes=False),
    scratch_types=[
        pltpu.VMEM((N_NODES + L,), jnp.int32),
        pltpu.VMEM((EPT + 8,), jnp.int32),
        pltpu.VMEM((EPT + 8,), jnp.int32),
        pltpu.VMEM((COMP,), jnp.int32),
        pltpu.VMEM((COMP,), jnp.int32),
        pltpu.VMEM((COMP,), jnp.int32),
        pltpu.VMEM((((NITER + L - 1) // L) * L,), jnp.int32),
        pltpu.VMEM((2 * PCH,), jnp.int32),
        pltpu.VMEM((2 * PCH,), jnp.int32),
        pltpu.VMEM((OWN_CAP,), jnp.int32),
        pltpu.VMEM((OWN_CAP,), jnp.int32),
        pltpu.VMEM((K,), jnp.int32),
        pltpu.VMEM((K, D_FEAT), jnp.float32),
        pltpu.VMEM((NS * L,), jnp.int32),
        pltpu.VMEM((L,), jnp.int32),
        pltpu.VMEM((ACC_SEG, ROWW), jnp.float32),
        pltpu.VMEM((L,), jnp.int32),
        pltpu.VMEM((L,), jnp.int32),
        pltpu.VMEM((L, ROWW), jnp.float32),
        pltpu.VMEM((L, ROWW), jnp.float32),
        pltpu.VMEM((L, D_FEAT), jnp.float32),
        pltpu.VMEM_SHARED((NS * COMP,), jnp.int32),
        pltpu.VMEM_SHARED((NS * COMP,), jnp.int32),
        pltpu.VMEM_SHARED((NS * L,), jnp.int32),
        pltpu.SemaphoreType.DMA,
        pltpu.SemaphoreType.DMA,
        pltpu.SemaphoreType.REGULAR,
    ],
)

@jax.jit
def kernel(all_node_emb, supernode_edge_index, supernode_idx, graph_batch):
    emb = all_node_emb.astype(jnp.float32)
    src = supernode_edge_index[0].astype(jnp.int32)
    dst = supernode_edge_index[1].astype(jnp.int32)
    snode = supernode_idx.astype(jnp.int32)

    remap = jnp.full((N_NODES + L,), -1, jnp.int32).at[snode].set(
        jnp.arange(N_SUPER, dtype=jnp.int32))
    canon = remap[snode]

    zacc = jnp.zeros((ACC_SEG, ROWW), jnp.float32)

    _, out = _main(emb, src, dst, remap, zacc, canon)
    return out
